# baseline (speedup 1.0000x reference)
import jax
import jax.numpy as jnp
from jax.experimental import pallas as pl
from jax.experimental.pallas import tpu as pltpu


def kernel(x, row_w, col_w):
    n, dim, h, w = x.shape
    half = dim // 2
    hw = h * w

    def body(row_ref, col_ref, out_ref, buf, sem):
        lane = jax.lax.broadcasted_iota(jnp.int32, (w, hw), 1)
        src = jax.lax.broadcasted_iota(jnp.int32, (w, hw), 0)
        p = (lane % w == src).astype(jnp.float32)
        xe = jax.lax.dot_general(
            col_ref[...], p, (((0,), (0,)), ((), ())),
            preferred_element_type=jnp.float32,
        )
        buf[0:half, :] = xe
        copies = []
        for k in range(n):
            cp = pltpu.make_async_copy(
                buf.at[pl.ds(0, half)],
                out_ref.at[k, pl.ds(0, half)],
                sem.at[k],
            )
            cp.start()
            copies.append(cp)
        lane_h = jax.lax.broadcasted_iota(jnp.int32, (h, hw), 1)
        src_h = jax.lax.broadcasted_iota(jnp.int32, (h, hw), 0)
        q = (lane_h // w == src_h).astype(jnp.float32)
        ye = jax.lax.dot_general(
            row_ref[...], q, (((0,), (0,)), ((), ())),
            preferred_element_type=jnp.float32,
        )
        buf[half:dim, :] = ye
        for k in range(n):
            cp = pltpu.make_async_copy(
                buf.at[pl.ds(half, half)],
                out_ref.at[k, pl.ds(half, half)],
                sem.at[k],
            )
            cp.start()
            copies.append(cp)
        for cp in copies:
            cp.wait()

    out = pl.pallas_call(
        body,
        in_specs=[
            pl.BlockSpec(memory_space=pltpu.VMEM),
            pl.BlockSpec(memory_space=pltpu.VMEM),
        ],
        out_specs=pl.BlockSpec(memory_space=pl.ANY),
        out_shape=jax.ShapeDtypeStruct((n, dim, hw), jnp.float32),
        scratch_shapes=[
            pltpu.VMEM((dim, hw), jnp.float32),
            pltpu.SemaphoreType.DMA((n,)),
        ],
    )(row_w[:h], col_w[:w])
    return out.reshape(n, dim, h, w)
